# baseline (speedup 1.0000x reference)
import functools

import jax
import jax.numpy as jnp
from jax import lax
from jax.experimental import pallas as pl
from jax.experimental.pallas import tpu as pltpu
from jax.experimental.pallas import tpu_sc as plsc

_NC = 2
_NS = 16
_NW = _NC * _NS

_B = 4096
_L = 50
_H = 32
_TOT = _B * _L
_PER_W = _TOT // _NW
_CH = 128
_NCH = _PER_W // _CH


def _emb_body(cond_hbm, table_hbm, out_hbm, idx_v, rows_v, sem):
    wid = lax.axis_index("s") * _NC + lax.axis_index("c")
    pltpu.sync_copy(cond_hbm.at[wid], idx_v)

    def step(j, carry):
        pltpu.async_copy(table_hbm.at[idx_v.at[j]], rows_v, sem).wait()
        pltpu.sync_copy(rows_v, out_hbm.at[wid].at[j])
        return carry

    lax.fori_loop(0, _NCH, step, 0)


@jax.jit
def kernel(conditions, table):
    idx = conditions.reshape(_NW, _NCH, _CH)
    mesh = plsc.VectorSubcoreMesh(
        core_axis_name="c", subcore_axis_name="s",
        num_cores=_NC, num_subcores=_NS)
    out = pl.kernel(
        _emb_body,
        out_type=jax.ShapeDtypeStruct((_NW, _NCH, _CH, _H), jnp.float32),
        mesh=mesh,
        scratch_types=[
            pltpu.VMEM((_NCH, _CH), jnp.int32),
            pltpu.VMEM((_CH, _H), jnp.float32),
            pltpu.SemaphoreType.DMA,
        ],
        compiler_params=pltpu.CompilerParams(use_tc_tiling_on_sc=False),
    )(idx, table)
    return out.reshape(_B, _L * _H)


# speedup vs baseline: 1.0614x; 1.0614x over previous
import jax
import jax.numpy as jnp
from jax import lax
from jax.experimental import pallas as pl
from jax.experimental.pallas import tpu as pltpu
from jax.experimental.pallas import tpu_sc as plsc

_NC = 2
_NS = 16
_NW = _NC * _NS

_B = 4096
_L = 50
_H = 32
_TOT = _B * _L
_PER_W = _TOT // _NW
_CH = 128
_NCH = _PER_W // _CH
_SUP = 5
_NSUP = _NCH // _SUP
_SROWS = _SUP * _CH


def _emb_body(cond_hbm, table_hbm, out_hbm, idx_v, buf0, buf1, g0, g1, w0, w1):
    wid = lax.axis_index("s") * _NC + lax.axis_index("c")
    out_w = out_hbm.at[wid]
    pltpu.sync_copy(cond_hbm.at[wid], idx_v)

    def fire(buf, gsem, s):
        for k in range(_SUP):
            pltpu.async_copy(
                table_hbm.at[idx_v.at[s * _SUP + k]],
                buf.at[pl.ds(k * _CH, _CH)], gsem)

    def drain_gather(buf, gsem, s):
        pltpu.make_async_copy(out_w.at[s], buf, gsem).wait()

    def start_write(buf, wsem, s):
        pltpu.async_copy(buf, out_w.at[s], wsem)

    def wait_write(buf, wsem, s):
        pltpu.make_async_copy(buf, out_w.at[s], wsem).wait()

    def outer(t, carry):
        s0 = 2 * t
        s1 = 2 * t + 1

        @pl.when(t > 0)
        def _():
            wait_write(buf0, w0, s0 - 2)
        fire(buf0, g0, s0)

        @pl.when(t > 0)
        def _():
            drain_gather(buf1, g1, s0 - 1)
            start_write(buf1, w1, s0 - 1)
            wait_write(buf1, w1, s1 - 2)
        fire(buf1, g1, s1)

        drain_gather(buf0, g0, s0)
        start_write(buf0, w0, s0)
        return carry

    lax.fori_loop(0, _NSUP // 2, outer, 0)
    drain_gather(buf1, g1, _NSUP - 1)
    start_write(buf1, w1, _NSUP - 1)
    wait_write(buf0, w0, _NSUP - 2)
    wait_write(buf1, w1, _NSUP - 1)


@jax.jit
def kernel(conditions, table):
    idx = conditions.reshape(_NW, _NCH, _CH)
    mesh = plsc.VectorSubcoreMesh(
        core_axis_name="c", subcore_axis_name="s",
        num_cores=_NC, num_subcores=_NS)
    out = pl.kernel(
        _emb_body,
        out_type=jax.ShapeDtypeStruct((_NW, _NSUP, _SROWS, _H), jnp.float32),
        mesh=mesh,
        scratch_types=[
            pltpu.VMEM((_NCH, _CH), jnp.int32),
            pltpu.VMEM((_SROWS, _H), jnp.float32),
            pltpu.VMEM((_SROWS, _H), jnp.float32),
            pltpu.SemaphoreType.DMA,
            pltpu.SemaphoreType.DMA,
            pltpu.SemaphoreType.DMA,
            pltpu.SemaphoreType.DMA,
        ],
        compiler_params=pltpu.CompilerParams(use_tc_tiling_on_sc=False),
    )(idx, table)
    return out.reshape(_B, _L * _H)
